# Initial kernel scaffold; baseline (speedup 1.0000x reference)
import jax
import jax.numpy as jnp
from jax.experimental import pallas as pl


def kernel(x, segment_ids):
    raise NotImplementedError("write your pallas kernel here")


# speedup vs baseline: 11.6022x; 11.6022x over previous
import jax
import jax.numpy as jnp
from jax import lax
from jax.experimental import pallas as pl

N_NODES_K = 50000
D_K = 256
G_K = 64
EPS_K = 1e-6

ROWS_BLK = 1000
N_BLKS = N_NODES_K // ROWS_BLK


def _sums_body(x_ref, seg_ref, s1_ref, s2_ref, cnt_ref):
    i = pl.program_id(0)

    @pl.when(i == 0)
    def _init():
        s1_ref[...] = jnp.zeros_like(s1_ref)
        s2_ref[...] = jnp.zeros_like(s2_ref)
        cnt_ref[...] = jnp.zeros_like(cnt_ref)

    x = x_ref[...]
    seg = seg_ref[0, 0, :]
    oh_t = (lax.broadcasted_iota(jnp.int32, (G_K, ROWS_BLK), 0) == seg[None, :]).astype(jnp.float32)
    s1_ref[...] += jnp.dot(oh_t, x, preferred_element_type=jnp.float32)
    s2_ref[...] += jnp.dot(oh_t, x * x, preferred_element_type=jnp.float32)
    cnt_ref[...] += jnp.broadcast_to(jnp.sum(oh_t, axis=1, keepdims=True), (G_K, 128))


def _apply_body(x_ref, seg_ref, s1_ref, s2_ref, cnt_ref, out_ref):
    cnt = jnp.maximum(cnt_ref[:, 0:1], 1.0)
    inv = 1.0 / cnt
    mu = s1_ref[...] * inv
    var = s2_ref[...] * inv - mu * mu
    b = mu * lax.rsqrt(var + EPS_K)

    seg = seg_ref[0, 0, :]
    oh = (seg[:, None] == lax.broadcasted_iota(jnp.int32, (ROWS_BLK, G_K), 1)).astype(jnp.float32)
    out_ref[...] = x_ref[...] - jnp.dot(oh, b, preferred_element_type=jnp.float32)


def kernel(x, segment_ids):
    seg = segment_ids.astype(jnp.int32).reshape(N_BLKS, 1, ROWS_BLK)

    s1, s2, cnt = pl.pallas_call(
        _sums_body,
        grid=(N_BLKS,),
        in_specs=[
            pl.BlockSpec((ROWS_BLK, D_K), lambda i: (i, 0)),
            pl.BlockSpec((1, 1, ROWS_BLK), lambda i: (i, 0, 0)),
        ],
        out_specs=[
            pl.BlockSpec((G_K, D_K), lambda i: (0, 0)),
            pl.BlockSpec((G_K, D_K), lambda i: (0, 0)),
            pl.BlockSpec((G_K, 128), lambda i: (0, 0)),
        ],
        out_shape=[
            jax.ShapeDtypeStruct((G_K, D_K), jnp.float32),
            jax.ShapeDtypeStruct((G_K, D_K), jnp.float32),
            jax.ShapeDtypeStruct((G_K, 128), jnp.float32),
        ],
    )(x, seg)

    out = pl.pallas_call(
        _apply_body,
        grid=(N_BLKS,),
        in_specs=[
            pl.BlockSpec((ROWS_BLK, D_K), lambda i: (i, 0)),
            pl.BlockSpec((1, 1, ROWS_BLK), lambda i: (i, 0, 0)),
            pl.BlockSpec((G_K, D_K), lambda i: (0, 0)),
            pl.BlockSpec((G_K, D_K), lambda i: (0, 0)),
            pl.BlockSpec((G_K, 128), lambda i: (0, 0)),
        ],
        out_specs=pl.BlockSpec((ROWS_BLK, D_K), lambda i: (i, 0)),
        out_shape=jax.ShapeDtypeStruct((N_NODES_K, D_K), jnp.float32),
    )(x, seg, s1, s2, cnt)

    return out
